# baseline (speedup 1.0000x reference)
import jax
import jax.numpy as jnp
from jax import lax
from jax.experimental import pallas as pl
from jax.experimental.pallas import tpu as pltpu
from jax.experimental.pallas import tpu_sc as plsc

N_NODES = 10000
N_EDGES = 320000
D = 128

NC = 2
NS = 16
NW = NC * NS
E_W = N_EDGES // NW
K = 80
NCHUNK = E_W // K
NBUF = 4
DR = 80
NDRAIN = N_NODES // DR
DRAIN_ITERS = -(-NDRAIN // NS)


def _sc_aggregate(x, src, dst, adj):
    mesh = plsc.VectorSubcoreMesh(core_axis_name="c", subcore_axis_name="s")

    def body(x_h, src_h, dst_h, adj_h, part_h,
             acc, rows_, srcv_, dstv_, adjv_,
             gsem_, ssem_, asem_, csem_):
        c = lax.axis_index("c")
        s = lax.axis_index("s")
        wid = c * NS + s

        zbuf = rows_[0]

        def zb(i, carry):
            for j in range(D // 16):
                zbuf[i, pl.ds(16 * j, 16)] = jnp.zeros((16,), jnp.float32)
            return carry
        lax.fori_loop(0, DR, zb, 0)
        for i in range(DRAIN_ITERS):
            ci = i * NS + s

            @pl.when(ci < NDRAIN)
            def _zero():
                r = pl.multiple_of(ci * DR, 8)
                pltpu.sync_copy(zbuf, acc.at[pl.ds(r, DR)])
        plsc.subcore_barrier()

        base0 = wid * E_W

        def issue_src(k, b):
            pltpu.async_copy(src_h.at[pl.ds(base0 + k * K, K)],
                             srcv_[b], ssem_[b])

        def wait_src(k, b):
            pltpu.make_async_copy(src_h.at[pl.ds(base0 + k * K, K)],
                                  srcv_[b], ssem_[b]).wait()

        def issue_da(k, b):
            pltpu.async_copy(dst_h.at[pl.ds(base0 + k * K, K)],
                             dstv_[b], asem_[b])
            pltpu.async_copy(adj_h.at[pl.ds(base0 + k * K, K)],
                             adjv_[b], asem_[b])

        def wait_da(k, b):
            pltpu.make_async_copy(dst_h.at[pl.ds(base0 + k * K, K)],
                                  dstv_[b], asem_[b]).wait()
            pltpu.make_async_copy(adj_h.at[pl.ds(base0 + k * K, K)],
                                  adjv_[b], asem_[b]).wait()

        def issue_gather(b):
            pltpu.async_copy(x_h.at[srcv_[b]], rows_[b], gsem_[b])

        def wait_gather(b):
            pltpu.make_async_copy(x_h.at[srcv_[b]], rows_[b], gsem_[b]).wait()

        def scale_rows(b):
            rows = rows_[b]
            adjv = adjv_[b]

            def scale(g, inner):
                avec = adjv[pl.ds(16 * g, 16)]
                for l in range(16):
                    a = avec[l]
                    e = 16 * g + l
                    for j in range(D // 16):
                        sl = pl.ds(16 * j, 16)
                        rows[e, sl] = rows[e, sl] * a
                return inner
            lax.fori_loop(0, K // 16, scale, 0)

        def issue_scatter(b):
            pltpu.async_copy(rows_[b], acc.at[dstv_[b]], csem_[b], add=True)

        def wait_scatter(b):
            pltpu.make_async_copy(rows_[b], acc.at[dstv_[b]], csem_[b]).wait()

        def step(k, b, scwait, pref, gath):
            b1 = (b + 1) % NBUF
            b2 = (b + 2) % NBUF
            wait_gather(b)
            wait_da(k, b)
            scale_rows(b)
            issue_scatter(b)
            if scwait:
                wait_scatter(b2)
            if pref:
                issue_src(k + 2, b2)
                issue_da(k + 2, b2)
            if gath:
                wait_src(k + 1, b1)
                issue_gather(b1)

        issue_src(0, 0)
        issue_da(0, 0)
        issue_src(1, 1)
        issue_da(1, 1)
        wait_src(0, 0)
        issue_gather(0)

        step(0, 0, False, True, True)
        step(1, 1, False, True, True)

        def quad(i, carry):
            for j in range(NBUF):
                step(2 + 4 * i + j, (2 + j) % NBUF, True, True, True)
            return carry
        lax.fori_loop(0, (NCHUNK - 5) // NBUF, quad, 0)
        step(NCHUNK - 3, (NCHUNK - 3) % NBUF, True, True, True)
        step(NCHUNK - 2, (NCHUNK - 2) % NBUF, True, False, True)
        step(NCHUNK - 1, (NCHUNK - 1) % NBUF, True, False, False)
        wait_scatter((NCHUNK - 2) % NBUF)
        wait_scatter((NCHUNK - 1) % NBUF)

        plsc.subcore_barrier()
        for i in range(DRAIN_ITERS):
            ci = i * NS + s

            @pl.when(ci < NDRAIN)
            def _drain():
                r = pl.multiple_of(ci * DR, 8)
                ro = pl.multiple_of(c * N_NODES + ci * DR, 8)
                pltpu.sync_copy(acc.at[pl.ds(r, DR)],
                                part_h.at[pl.ds(ro, DR)])

    run = pl.kernel(
        body,
        out_type=jax.ShapeDtypeStruct((NC * N_NODES, D), jnp.float32),
        mesh=mesh,
        scratch_types=[
            pltpu.VMEM_SHARED((N_NODES, D), jnp.float32),
            [pltpu.VMEM((K, D), jnp.float32) for _ in range(NBUF)],
            [pltpu.VMEM((K,), jnp.int32) for _ in range(NBUF)],
            [pltpu.VMEM((K,), jnp.int32) for _ in range(NBUF)],
            [pltpu.VMEM((K,), jnp.float32) for _ in range(NBUF)],
            [pltpu.SemaphoreType.DMA for _ in range(NBUF)],
            [pltpu.SemaphoreType.DMA for _ in range(NBUF)],
            [pltpu.SemaphoreType.DMA for _ in range(NBUF)],
            [pltpu.SemaphoreType.DMA for _ in range(NBUF)],
        ],
    )
    return run(x, src, dst, adj)


def _tc_combine_matmul(part, W):
    BR = 1000

    def mm(p0_ref, p1_ref, w_ref, o_ref):
        o_ref[...] = jnp.dot(p0_ref[...] + p1_ref[...], w_ref[...],
                             preferred_element_type=jnp.float32)

    nblk = N_NODES // BR
    return pl.pallas_call(
        mm,
        grid=(nblk,),
        in_specs=[
            pl.BlockSpec((BR, D), lambda i: (i, 0)),
            pl.BlockSpec((BR, D), lambda i: (i + nblk, 0)),
            pl.BlockSpec((D, D), lambda i: (0, 0)),
        ],
        out_specs=pl.BlockSpec((BR, D), lambda i: (i, 0)),
        out_shape=jax.ShapeDtypeStruct((N_NODES, D), jnp.float32),
    )(part, part, W)


def kernel(x, edge_index, adj_values, W):
    ei = edge_index.astype(jnp.int32)
    dst = ei[0]
    src = ei[1]
    part = _sc_aggregate(x, src, dst, adj_values)
    return _tc_combine_matmul(part, W)


# speedup vs baseline: 1.3695x; 1.3695x over previous
import jax
import jax.numpy as jnp
from jax import lax
from jax.experimental import pallas as pl
from jax.experimental.pallas import tpu as pltpu
from jax.experimental.pallas import tpu_sc as plsc

N_NODES = 10000
N_EDGES = 320000
D = 128

NC = 2
NS = 16
NW = NC * NS
E_W = N_EDGES // NW
K = 80
NCHUNK = E_W // K
NBUF = 2
DR = 80
NDRAIN = N_NODES // DR
DRAIN_ITERS = -(-NDRAIN // NS)


def _sc_aggregate(x, src, dst, adj):
    mesh = plsc.VectorSubcoreMesh(core_axis_name="c", subcore_axis_name="s")

    def body(x_h, src_h, dst_h, adj_h, part_h,
             acc, rows_, srcv_, dstv_, adjv_,
             gsem_, ssem_, asem_):
        c = lax.axis_index("c")
        s = lax.axis_index("s")
        wid = c * NS + s

        zbuf = rows_[0]

        def zb(i, carry):
            for j in range(D // 16):
                zbuf[i, pl.ds(16 * j, 16)] = jnp.zeros((16,), jnp.float32)
            return carry
        lax.fori_loop(0, DR, zb, 0)
        for i in range(DRAIN_ITERS):
            ci = i * NS + s

            @pl.when(ci < NDRAIN)
            def _zero():
                r = pl.multiple_of(ci * DR, 8)
                pltpu.sync_copy(zbuf, acc.at[pl.ds(r, DR)])
        plsc.subcore_barrier()

        base0 = wid * E_W

        def issue_src(k, b):
            pltpu.async_copy(src_h.at[pl.ds(base0 + k * K, K)],
                             srcv_[b], ssem_[b])

        def wait_src(k, b):
            pltpu.make_async_copy(src_h.at[pl.ds(base0 + k * K, K)],
                                  srcv_[b], ssem_[b]).wait()

        def issue_da(k, b):
            pltpu.async_copy(dst_h.at[pl.ds(base0 + k * K, K)],
                             dstv_[b], asem_[b])
            pltpu.async_copy(adj_h.at[pl.ds(base0 + k * K, K)],
                             adjv_[b], asem_[b])

        def wait_da(k, b):
            pltpu.make_async_copy(dst_h.at[pl.ds(base0 + k * K, K)],
                                  dstv_[b], asem_[b]).wait()
            pltpu.make_async_copy(adj_h.at[pl.ds(base0 + k * K, K)],
                                  adjv_[b], asem_[b]).wait()

        def issue_gather(b):
            pltpu.async_copy(x_h.at[srcv_[b]], rows_[b], gsem_[b])

        def wait_gather(b):
            pltpu.make_async_copy(x_h.at[srcv_[b]], rows_[b], gsem_[b]).wait()

        def scale_rows(b):
            rows = rows_[b]
            adjv = adjv_[b]

            def scale(g, inner):
                avec = adjv[pl.ds(16 * g, 16)]
                for l in range(16):
                    a = avec[l]
                    e = 16 * g + l
                    for j in range(D // 16):
                        sl = pl.ds(16 * j, 16)
                        rows[e, sl] = rows[e, sl] * a
                return inner
            lax.fori_loop(0, K // 16, scale, 0)

        def process(k, b, do_prefetch):
            kp = k + 2
            wait_gather(b)
            if do_prefetch:
                @pl.when(kp < NCHUNK)
                def _psrc():
                    issue_src(kp, b)
            wait_da(k, b)
            scale_rows(b)
            pltpu.sync_copy(rows_[b], acc.at[dstv_[b]], add=True)
            if do_prefetch:
                @pl.when(kp < NCHUNK)
                def _pnext():
                    issue_da(kp, b)
                    wait_src(kp, b)
                    issue_gather(b)

        for b in range(2):
            issue_src(b, b)
            issue_da(b, b)
            wait_src(b, b)
            issue_gather(b)

        def pair(i, carry):
            process(2 * i, 0, True)
            process(2 * i + 1, 1, True)
            return carry
        lax.fori_loop(0, NCHUNK // 2, pair, 0)
        process(NCHUNK - 1, 0, False)

        plsc.subcore_barrier()
        for i in range(DRAIN_ITERS):
            ci = i * NS + s

            @pl.when(ci < NDRAIN)
            def _drain():
                r = pl.multiple_of(ci * DR, 8)
                ro = pl.multiple_of(c * N_NODES + ci * DR, 8)
                pltpu.sync_copy(acc.at[pl.ds(r, DR)],
                                part_h.at[pl.ds(ro, DR)])

    run = pl.kernel(
        body,
        out_type=jax.ShapeDtypeStruct((NC * N_NODES, D), jnp.float32),
        mesh=mesh,
        scratch_types=[
            pltpu.VMEM_SHARED((N_NODES, D), jnp.float32),
            [pltpu.VMEM((K, D), jnp.float32) for _ in range(NBUF)],
            [pltpu.VMEM((K,), jnp.int32) for _ in range(NBUF)],
            [pltpu.VMEM((K,), jnp.int32) for _ in range(NBUF)],
            [pltpu.VMEM((K,), jnp.float32) for _ in range(NBUF)],
            [pltpu.SemaphoreType.DMA for _ in range(NBUF)],
            [pltpu.SemaphoreType.DMA for _ in range(NBUF)],
            [pltpu.SemaphoreType.DMA for _ in range(NBUF)],
        ],
    )
    return run(x, src, dst, adj)


def _tc_combine_matmul(part, W):
    BR = 1000

    def mm(p0_ref, p1_ref, w_ref, o_ref):
        o_ref[...] = jnp.dot(p0_ref[...] + p1_ref[...], w_ref[...],
                             preferred_element_type=jnp.float32)

    nblk = N_NODES // BR
    return pl.pallas_call(
        mm,
        grid=(nblk,),
        in_specs=[
            pl.BlockSpec((BR, D), lambda i: (i, 0)),
            pl.BlockSpec((BR, D), lambda i: (i + nblk, 0)),
            pl.BlockSpec((D, D), lambda i: (0, 0)),
        ],
        out_specs=pl.BlockSpec((BR, D), lambda i: (i, 0)),
        out_shape=jax.ShapeDtypeStruct((N_NODES, D), jnp.float32),
    )(part, part, W)


def kernel(x, edge_index, adj_values, W):
    ei = edge_index.astype(jnp.int32)
    dst = ei[0]
    src = ei[1]
    part = _sc_aggregate(x, src, dst, adj_values)
    return _tc_combine_matmul(part, W)
